# baseline (speedup 1.0000x reference)
import functools

import jax
import jax.numpy as jnp
from jax import lax
from jax.experimental import pallas as pl
from jax.experimental.pallas import tpu as pltpu
from jax.experimental.pallas import tpu_sc as plsc

COMMITMENT_COST = 0.25


def _segments(k):
    if k == 8192:
        return (0, 2736, 5472, 8192)
    return (0, k)


def _bf16(v):
    return v.astype(jnp.bfloat16).astype(jnp.float32)


def _argmin_body(segs, bn, x_ref, xn_ref, ct_ref, cn_ref, idx_ref, mind_ref):
    x = x_ref[...]
    x2 = x + x
    xn = xn_ref[...]
    bm = x.shape[0]
    k = ct_ref.shape[1]
    nchunk = k // bn
    nseg = len(segs) - 1
    best = [jnp.full((bm, 1), jnp.inf, jnp.float32) for _ in range(nseg)]
    bidx = [jnp.zeros((bm, 1), jnp.float32) for _ in range(nseg)]
    io = lax.broadcasted_iota(jnp.int32, (bm, bn), 1).astype(jnp.float32)
    io1 = lax.broadcasted_iota(jnp.int32, (1, bn), 1)
    for j in range(nchunk):
        lo, hi = j * bn, (j + 1) * bn
        ct = ct_ref[:, lo:hi]
        s2 = lax.dot_general(x2, ct, (((1,), (0,)), ((), ())),
                             preferred_element_type=jnp.float32)
        cn = cn_ref[:, lo:hi]
        d = (xn + cn) - s2
        for si in range(nseg):
            slo, shi = segs[si], segs[si + 1]
            if shi <= lo or slo >= hi:
                continue
            if slo <= lo and hi <= shi:
                dm = d
            else:
                gio = io1 + lo
                inseg = (gio >= slo) & (gio < shi)
                bias = jnp.where(inseg, 0.0, jnp.inf)
                dm = d + bias
            m = jnp.min(dm, axis=1, keepdims=True)
            cand = jnp.min(jnp.where(dm == m, io, jnp.float32(3e9)),
                           axis=1, keepdims=True) + jnp.float32(lo)
            take = m < best[si]
            bidx[si] = jnp.where(take, cand, bidx[si])
            best[si] = jnp.where(take, m, best[si])
    acc_cmp = _bf16(best[0])
    acc_idx = bidx[0]
    acc_true = best[0]
    for si in range(1, nseg):
        take = best[si] < acc_cmp
        acc_idx = jnp.where(take, bidx[si], acc_idx)
        acc_true = jnp.where(take, best[si], acc_true)
        acc_cmp = _bf16(jnp.where(take, best[si], acc_cmp))
    idx_ref[...] = acc_idx.astype(jnp.int32)
    mind_ref[...] = acc_true


def _distance_argmin(flat, xn, ct, cn, bm=256, bn=1024):
    n, d = flat.shape
    k = ct.shape[1]
    segs = _segments(k)
    grid = (n // bm,)
    body = functools.partial(_argmin_body, segs, bn)
    idx2, mind = pl.pallas_call(
        body,
        grid=grid,
        in_specs=[
            pl.BlockSpec((bm, d), lambda i: (i, 0)),
            pl.BlockSpec((bm, 1), lambda i: (i, 0)),
            pl.BlockSpec((d, k), lambda i: (0, 0)),
            pl.BlockSpec((1, k), lambda i: (0, 0)),
        ],
        out_specs=[
            pl.BlockSpec((bm, 1), lambda i: (i, 0)),
            pl.BlockSpec((bm, 1), lambda i: (i, 0)),
        ],
        out_shape=[
            jax.ShapeDtypeStruct((n, 1), jnp.int32),
            jax.ShapeDtypeStruct((n, 1), jnp.float32),
        ],
        compiler_params=pltpu.CompilerParams(
            dimension_semantics=("parallel",),
        ),
    )(flat, xn, ct, cn)
    return idx2[:, 0], mind[:, 0]


def _sc_gather_hist(codebook, idx, zeros_hist):
    k, d = codebook.shape
    n = idx.shape[0]
    info = plsc.get_sparse_core_info()
    nc, ns = info.num_cores, info.num_subcores
    nw = nc * ns
    rpw = n // nw
    c = 128
    nch = rpw // c
    ks = k // ns
    idx2 = idx.reshape(n // c, c)

    mesh = plsc.VectorSubcoreMesh(core_axis_name="c", subcore_axis_name="s")

    @functools.partial(
        pl.kernel, mesh=mesh,
        out_type=[jax.ShapeDtypeStruct((n, d), jnp.float32),
                  jax.ShapeDtypeStruct((nc, k, 16), jnp.float32)],
        scratch_types=[
            pltpu.VMEM((nch, c), jnp.int32),
            pltpu.VMEM((c, d), jnp.float32),
            pltpu.VMEM((c, 16), jnp.float32),
            pltpu.VMEM_SHARED((k, 16), jnp.float32),
            pltpu.SemaphoreType.DMA,
        ],
    )
    def sck(cb_hbm, idx_hbm, z_hbm, q_hbm, cnt_hbm, idx_v, rows_v, ones_v,
            cnt_sh, sem):
        cid = lax.axis_index("c")
        sid = lax.axis_index("s")
        wid = sid * nc + cid
        one = jnp.full((16,), 1.0, jnp.float32)

        def fill_ones(i, _):
            ones_v[i, :] = one
            return 0

        lax.fori_loop(0, c, fill_ones, 0)
        pltpu.sync_copy(idx_hbm.at[pl.ds(wid * nch, nch)], idx_v)
        pltpu.sync_copy(z_hbm.at[pl.ds(sid * ks, ks)],
                        cnt_sh.at[pl.ds(sid * ks, ks)])
        plsc.subcore_barrier()
        for j in range(nch):
            pltpu.async_copy(cb_hbm.at[idx_v.at[j]], rows_v, sem).wait()
            pltpu.sync_copy(rows_v, q_hbm.at[pl.ds((wid * nch + j) * c, c)])
            pltpu.sync_copy(ones_v, cnt_sh.at[idx_v.at[j]], add=True)
        plsc.subcore_barrier()
        pltpu.sync_copy(cnt_sh.at[pl.ds(sid * ks, ks)],
                        cnt_hbm.at[cid, pl.ds(sid * ks, ks)])

    q, cnt = sck(codebook, idx2, zeros_hist)
    counts = cnt[0, :, 0] + cnt[1, :, 0]
    return q, counts


def kernel(inputs, codebook):
    input_shape = inputs.shape
    d = input_shape[-1]
    k = codebook.shape[0]
    flat = inputs.reshape(-1, d)
    n = flat.shape[0]

    xn = jnp.sum(flat ** 2, axis=1, keepdims=True)
    cn = jnp.sum(codebook ** 2, axis=1)
    ct = codebook.T

    h = n // 2
    cnr = cn.reshape(1, k)
    zeros_hist = jnp.zeros((k, 16), jnp.float32)
    idx0, mind0 = _distance_argmin(flat[:h], xn[:h], ct, cnr)
    q0, counts0 = _sc_gather_hist(codebook, idx0, zeros_hist)
    idx1, mind1 = _distance_argmin(flat[h:], xn[h:], ct, cnr)
    q1, counts1 = _sc_gather_hist(codebook, idx1, zeros_hist)
    idx = jnp.concatenate([idx0, idx1])
    quantized = jnp.concatenate([q0, q1])
    counts = counts0 + counts1

    loss = COMMITMENT_COST * ((jnp.sum(mind0) + jnp.sum(mind1)) / (n * d))
    quantized_st = (flat + (quantized - flat)).reshape(input_shape)
    avg_probs = counts / n
    perplexity = jnp.exp(-jnp.sum(avg_probs * jnp.log(avg_probs + 1e-10)))
    return quantized_st, loss, perplexity, idx


# speedup vs baseline: 1.0980x; 1.0980x over previous
import functools

import jax
import jax.numpy as jnp
from jax import lax
from jax.experimental import pallas as pl
from jax.experimental.pallas import tpu as pltpu
from jax.experimental.pallas import tpu_sc as plsc

COMMITMENT_COST = 0.25


def _segments(k):
    if k == 8192:
        return (0, 2736, 5472, 8192)
    return (0, k)


def _bf16(v):
    return v.astype(jnp.bfloat16).astype(jnp.float32)


def _argmin_body(segs, bn, x_ref, xn_ref, ct_ref, cn_ref, idx_ref, mind_ref):
    x = x_ref[...]
    x2 = x + x
    xn = xn_ref[...]
    bm = x.shape[0]
    k = ct_ref.shape[1]
    nchunk = k // bn
    nseg = len(segs) - 1
    best = [jnp.full((bm, 1), jnp.inf, jnp.float32) for _ in range(nseg)]
    bidx = [jnp.zeros((bm, 1), jnp.float32) for _ in range(nseg)]
    io = lax.broadcasted_iota(jnp.int32, (bm, bn), 1).astype(jnp.float32)
    io1 = lax.broadcasted_iota(jnp.int32, (1, bn), 1)
    for j in range(nchunk):
        lo, hi = j * bn, (j + 1) * bn
        ct = ct_ref[:, lo:hi]
        s2 = lax.dot_general(x2, ct, (((1,), (0,)), ((), ())),
                             preferred_element_type=jnp.float32)
        cn = cn_ref[:, lo:hi]
        d = (xn + cn) - s2
        for si in range(nseg):
            slo, shi = segs[si], segs[si + 1]
            if shi <= lo or slo >= hi:
                continue
            if slo <= lo and hi <= shi:
                dm = d
            else:
                gio = io1 + lo
                inseg = (gio >= slo) & (gio < shi)
                bias = jnp.where(inseg, 0.0, jnp.inf)
                dm = d + bias
            m = jnp.min(dm, axis=1, keepdims=True)
            cand = jnp.min(jnp.where(dm == m, io, jnp.float32(3e9)),
                           axis=1, keepdims=True) + jnp.float32(lo)
            take = m < best[si]
            bidx[si] = jnp.where(take, cand, bidx[si])
            best[si] = jnp.where(take, m, best[si])
    acc_cmp = _bf16(best[0])
    acc_idx = bidx[0]
    acc_true = best[0]
    for si in range(1, nseg):
        take = best[si] < acc_cmp
        acc_idx = jnp.where(take, bidx[si], acc_idx)
        acc_true = jnp.where(take, best[si], acc_true)
        acc_cmp = _bf16(jnp.where(take, best[si], acc_cmp))
    idx_ref[...] = acc_idx.astype(jnp.int32)
    mind_ref[...] = acc_true


def _distance_argmin(flat, xn, ct, cn, bm=256, bn=1024):
    n, d = flat.shape
    k = ct.shape[1]
    segs = _segments(k)
    grid = (n // bm,)
    body = functools.partial(_argmin_body, segs, bn)
    idx2, mind = pl.pallas_call(
        body,
        grid=grid,
        in_specs=[
            pl.BlockSpec((bm, d), lambda i: (i, 0)),
            pl.BlockSpec((bm, 1), lambda i: (i, 0)),
            pl.BlockSpec((d, k), lambda i: (0, 0)),
            pl.BlockSpec((1, k), lambda i: (0, 0)),
        ],
        out_specs=[
            pl.BlockSpec((bm, 1), lambda i: (i, 0)),
            pl.BlockSpec((bm, 1), lambda i: (i, 0)),
        ],
        out_shape=[
            jax.ShapeDtypeStruct((n, 1), jnp.int32),
            jax.ShapeDtypeStruct((n, 1), jnp.float32),
        ],
        compiler_params=pltpu.CompilerParams(
            dimension_semantics=("parallel",),
        ),
    )(flat, xn, ct, cn)
    return idx2[:, 0], mind[:, 0]


def _sc_gather_hist(codebook, idx, zeros_hist):
    k, d = codebook.shape
    n = idx.shape[0]
    info = plsc.get_sparse_core_info()
    nc, ns = info.num_cores, info.num_subcores
    nw = nc * ns
    rpw = n // nw
    c = 128
    nch = rpw // c
    ks = k // ns
    idx2 = idx.reshape(n // c, c)

    mesh = plsc.VectorSubcoreMesh(core_axis_name="c", subcore_axis_name="s")

    @functools.partial(
        pl.kernel, mesh=mesh,
        out_type=[jax.ShapeDtypeStruct((n, d), jnp.float32),
                  jax.ShapeDtypeStruct((nc, k, 16), jnp.float32)],
        scratch_types=[
            pltpu.VMEM((nch, c), jnp.int32),
            pltpu.VMEM((c, d), jnp.float32),
            pltpu.VMEM((c, 16), jnp.float32),
            pltpu.VMEM_SHARED((k, 16), jnp.float32),
            pltpu.SemaphoreType.DMA,
        ],
    )
    def sck(cb_hbm, idx_hbm, z_hbm, q_hbm, cnt_hbm, idx_v, rows_a,
            ones_v, cnt_sh, gs_a):
        cid = lax.axis_index("c")
        sid = lax.axis_index("s")
        wid = sid * nc + cid
        one = jnp.full((16,), 1.0, jnp.float32)

        def fill_ones(i, _):
            ones_v[i, :] = one
            return 0

        lax.fori_loop(0, c, fill_ones, 0)
        pltpu.sync_copy(idx_hbm.at[pl.ds(wid * nch, nch)], idx_v)
        pltpu.sync_copy(z_hbm.at[pl.ds(sid * ks, ks)],
                        cnt_sh.at[pl.ds(sid * ks, ks)])
        plsc.subcore_barrier()
        for j in range(nch):
            pltpu.async_copy(cb_hbm.at[idx_v.at[j]], rows_a, gs_a).wait()
            pltpu.sync_copy(rows_a, q_hbm.at[pl.ds((wid * nch + j) * c, c)])
            pltpu.sync_copy(ones_v, cnt_sh.at[idx_v.at[j]], add=True)
        plsc.subcore_barrier()
        pltpu.sync_copy(cnt_sh.at[pl.ds(sid * ks, ks)],
                        cnt_hbm.at[cid, pl.ds(sid * ks, ks)])

    q, cnt = sck(codebook, idx2, zeros_hist)
    counts = cnt[0, :, 0] + cnt[1, :, 0]
    return q, counts


def kernel(inputs, codebook):
    input_shape = inputs.shape
    d = input_shape[-1]
    k = codebook.shape[0]
    flat = inputs.reshape(-1, d)
    n = flat.shape[0]

    xn = jnp.sum(flat ** 2, axis=1, keepdims=True)
    cn = jnp.sum(codebook ** 2, axis=1)
    ct = codebook.T

    idx, mind = _distance_argmin(flat, xn, ct, cn.reshape(1, k))

    zeros_hist = jnp.zeros((k, 16), jnp.float32)
    quantized, counts = _sc_gather_hist(codebook, idx, zeros_hist)

    loss = COMMITMENT_COST * (jnp.sum(mind) / (n * d))
    quantized_st = (flat + (quantized - flat)).reshape(input_shape)
    avg_probs = counts / n
    perplexity = jnp.exp(-jnp.sum(avg_probs * jnp.log(avg_probs + 1e-10)))
    return quantized_st, loss, perplexity, idx


# speedup vs baseline: 1.1509x; 1.0481x over previous
import functools

import jax
import jax.numpy as jnp
from jax import lax
from jax.experimental import pallas as pl
from jax.experimental.pallas import tpu as pltpu
from jax.experimental.pallas import tpu_sc as plsc

COMMITMENT_COST = 0.25


def _segments(k):
    if k == 8192:
        return (0, 2736, 5472, 8192)
    return (0, k)


def _bf16(v):
    return v.astype(jnp.bfloat16).astype(jnp.float32)


def _argmin_body(segs, bn, x_ref, xn_ref, ct_ref, cn_ref, idx_ref, mind_ref):
    x = x_ref[...]
    x2 = x + x
    xn = xn_ref[...]
    bm = x.shape[0]
    k = ct_ref.shape[1]
    nchunk = k // bn
    nseg = len(segs) - 1
    best = [jnp.full((bm, 1), jnp.inf, jnp.float32) for _ in range(nseg)]
    bidx = [jnp.zeros((bm, 1), jnp.float32) for _ in range(nseg)]
    io = lax.broadcasted_iota(jnp.int32, (bm, bn), 1).astype(jnp.float32)
    io1 = lax.broadcasted_iota(jnp.int32, (1, bn), 1)
    for j in range(nchunk):
        lo, hi = j * bn, (j + 1) * bn
        ct = ct_ref[:, lo:hi]
        s2 = lax.dot_general(x2, ct, (((1,), (0,)), ((), ())),
                             preferred_element_type=jnp.float32)
        cn = cn_ref[:, lo:hi]
        d = (xn + cn) - s2
        for si in range(nseg):
            slo, shi = segs[si], segs[si + 1]
            if shi <= lo or slo >= hi:
                continue
            if slo <= lo and hi <= shi:
                dm = d
            else:
                gio = io1 + lo
                inseg = (gio >= slo) & (gio < shi)
                bias = jnp.where(inseg, 0.0, jnp.inf)
                dm = d + bias
            m = jnp.min(dm, axis=1, keepdims=True)
            cand = jnp.min(jnp.where(dm == m, io, jnp.float32(3e9)),
                           axis=1, keepdims=True) + jnp.float32(lo)
            take = m < best[si]
            bidx[si] = jnp.where(take, cand, bidx[si])
            best[si] = jnp.where(take, m, best[si])
    acc_cmp = _bf16(best[0])
    acc_idx = bidx[0]
    acc_true = best[0]
    for si in range(1, nseg):
        take = best[si] < acc_cmp
        acc_idx = jnp.where(take, bidx[si], acc_idx)
        acc_true = jnp.where(take, best[si], acc_true)
        acc_cmp = _bf16(jnp.where(take, best[si], acc_cmp))
    idx_ref[...] = acc_idx.astype(jnp.int32)
    mind_ref[...] = acc_true


def _distance_argmin(flat, xn, ct, cn, bm=512, bn=1024):
    n, d = flat.shape
    k = ct.shape[1]
    segs = _segments(k)
    grid = (n // bm,)
    body = functools.partial(_argmin_body, segs, bn)
    idx2, mind = pl.pallas_call(
        body,
        grid=grid,
        in_specs=[
            pl.BlockSpec((bm, d), lambda i: (i, 0)),
            pl.BlockSpec((bm, 1), lambda i: (i, 0)),
            pl.BlockSpec((d, k), lambda i: (0, 0)),
            pl.BlockSpec((1, k), lambda i: (0, 0)),
        ],
        out_specs=[
            pl.BlockSpec((bm, 1), lambda i: (i, 0)),
            pl.BlockSpec((bm, 1), lambda i: (i, 0)),
        ],
        out_shape=[
            jax.ShapeDtypeStruct((n, 1), jnp.int32),
            jax.ShapeDtypeStruct((n, 1), jnp.float32),
        ],
        compiler_params=pltpu.CompilerParams(
            dimension_semantics=("parallel",),
        ),
    )(flat, xn, ct, cn)
    return idx2[:, 0], mind[:, 0]


def _sc_gather_hist(codebook, idx, zeros_hist):
    k, d = codebook.shape
    n = idx.shape[0]
    info = plsc.get_sparse_core_info()
    nc, ns = info.num_cores, info.num_subcores
    nw = nc * ns
    rpw = n // nw
    c = 128
    nch = rpw // c
    ks = k // ns
    idx2 = idx.reshape(n // c, c)

    mesh = plsc.VectorSubcoreMesh(core_axis_name="c", subcore_axis_name="s")

    @functools.partial(
        pl.kernel, mesh=mesh,
        out_type=[jax.ShapeDtypeStruct((n, d), jnp.float32),
                  jax.ShapeDtypeStruct((nc, k, 16), jnp.float32)],
        scratch_types=[
            pltpu.VMEM((nch, c), jnp.int32),
            pltpu.VMEM((c, d), jnp.float32),
            pltpu.VMEM((c, 16), jnp.float32),
            pltpu.VMEM_SHARED((k, 16), jnp.float32),
            pltpu.SemaphoreType.DMA,
        ],
    )
    def sck(cb_hbm, idx_hbm, z_hbm, q_hbm, cnt_hbm, idx_v, rows_a,
            ones_v, cnt_sh, gs_a):
        cid = lax.axis_index("c")
        sid = lax.axis_index("s")
        wid = sid * nc + cid
        one = jnp.full((16,), 1.0, jnp.float32)

        def fill_ones(i, _):
            ones_v[i, :] = one
            return 0

        lax.fori_loop(0, c, fill_ones, 0)
        pltpu.sync_copy(idx_hbm.at[pl.ds(wid * nch, nch)], idx_v)
        pltpu.sync_copy(z_hbm.at[pl.ds(sid * ks, ks)],
                        cnt_sh.at[pl.ds(sid * ks, ks)])
        plsc.subcore_barrier()
        for j in range(nch):
            pltpu.async_copy(cb_hbm.at[idx_v.at[j]], rows_a, gs_a).wait()
            pltpu.sync_copy(rows_a, q_hbm.at[pl.ds((wid * nch + j) * c, c)])
            pltpu.sync_copy(ones_v, cnt_sh.at[idx_v.at[j]], add=True)
        plsc.subcore_barrier()
        pltpu.sync_copy(cnt_sh.at[pl.ds(sid * ks, ks)],
                        cnt_hbm.at[cid, pl.ds(sid * ks, ks)])

    q, cnt = sck(codebook, idx2, zeros_hist)
    counts = cnt[0, :, 0] + cnt[1, :, 0]
    return q, counts


def kernel(inputs, codebook):
    input_shape = inputs.shape
    d = input_shape[-1]
    k = codebook.shape[0]
    flat = inputs.reshape(-1, d)
    n = flat.shape[0]

    xn = jnp.sum(flat ** 2, axis=1, keepdims=True)
    cn = jnp.sum(codebook ** 2, axis=1)
    ct = codebook.T

    idx, mind = _distance_argmin(flat, xn, ct, cn.reshape(1, k))

    zeros_hist = jnp.zeros((k, 16), jnp.float32)
    quantized, counts = _sc_gather_hist(codebook, idx, zeros_hist)

    loss = COMMITMENT_COST * (jnp.sum(mind) / (n * d))
    quantized_st = (flat + (quantized - flat)).reshape(input_shape)
    avg_probs = counts / n
    perplexity = jnp.exp(-jnp.sum(avg_probs * jnp.log(avg_probs + 1e-10)))
    return quantized_st, loss, perplexity, idx


# speedup vs baseline: 1.2137x; 1.0546x over previous
import functools

import jax
import jax.numpy as jnp
from jax import lax
from jax.experimental import pallas as pl
from jax.experimental.pallas import tpu as pltpu
from jax.experimental.pallas import tpu_sc as plsc

COMMITMENT_COST = 0.25


def _segments(k):
    if k == 8192:
        return (0, 2736, 5472, 8192)
    return (0, k)


def _bf16(v):
    return v.astype(jnp.bfloat16).astype(jnp.float32)


def _argmin_body(segs, bn, x_ref, xn_ref, ct_ref, cn_ref, idx_ref, mind_ref):
    x = x_ref[...]
    x2 = x + x
    xn = xn_ref[...]
    bm = x.shape[0]
    k = ct_ref.shape[1]
    nchunk = k // bn
    nseg = len(segs) - 1
    best = [jnp.full((bm, 1), jnp.inf, jnp.float32) for _ in range(nseg)]
    bidx = [jnp.zeros((bm, 1), jnp.float32) for _ in range(nseg)]
    io = lax.broadcasted_iota(jnp.int32, (bm, bn), 1).astype(jnp.float32)
    io1 = lax.broadcasted_iota(jnp.int32, (1, bn), 1)
    for j in range(nchunk):
        lo, hi = j * bn, (j + 1) * bn
        ct = ct_ref[:, lo:hi]
        s2 = lax.dot_general(x2, ct, (((1,), (0,)), ((), ())),
                             preferred_element_type=jnp.float32)
        cn = cn_ref[:, lo:hi]
        d = (xn + cn) - s2
        for si in range(nseg):
            slo, shi = segs[si], segs[si + 1]
            if shi <= lo or slo >= hi:
                continue
            if slo <= lo and hi <= shi:
                dm = d
            else:
                gio = io1 + lo
                inseg = (gio >= slo) & (gio < shi)
                bias = jnp.where(inseg, 0.0, jnp.inf)
                dm = d + bias
            m = jnp.min(dm, axis=1, keepdims=True)
            cand = jnp.min(jnp.where(dm == m, io, jnp.float32(3e9)),
                           axis=1, keepdims=True) + jnp.float32(lo)
            take = m < best[si]
            bidx[si] = jnp.where(take, cand, bidx[si])
            best[si] = jnp.where(take, m, best[si])
    acc_cmp = _bf16(best[0])
    acc_idx = bidx[0]
    acc_true = best[0]
    for si in range(1, nseg):
        take = best[si] < acc_cmp
        acc_idx = jnp.where(take, bidx[si], acc_idx)
        acc_true = jnp.where(take, best[si], acc_true)
        acc_cmp = _bf16(jnp.where(take, best[si], acc_cmp))
    idx_ref[...] = acc_idx.astype(jnp.int32)
    mind_ref[...] = acc_true


def _distance_argmin(flat, xn, ct, cn, bm=1024, bn=1024):
    n, d = flat.shape
    k = ct.shape[1]
    segs = _segments(k)
    grid = (n // bm,)
    body = functools.partial(_argmin_body, segs, bn)
    idx2, mind = pl.pallas_call(
        body,
        grid=grid,
        in_specs=[
            pl.BlockSpec((bm, d), lambda i: (i, 0)),
            pl.BlockSpec((bm, 1), lambda i: (i, 0)),
            pl.BlockSpec((d, k), lambda i: (0, 0)),
            pl.BlockSpec((1, k), lambda i: (0, 0)),
        ],
        out_specs=[
            pl.BlockSpec((bm, 1), lambda i: (i, 0)),
            pl.BlockSpec((bm, 1), lambda i: (i, 0)),
        ],
        out_shape=[
            jax.ShapeDtypeStruct((n, 1), jnp.int32),
            jax.ShapeDtypeStruct((n, 1), jnp.float32),
        ],
        compiler_params=pltpu.CompilerParams(
            dimension_semantics=("parallel",),
        ),
    )(flat, xn, ct, cn)
    return idx2[:, 0], mind[:, 0]


def _sc_gather_hist(codebook, idx, zeros_hist):
    k, d = codebook.shape
    n = idx.shape[0]
    info = plsc.get_sparse_core_info()
    nc, ns = info.num_cores, info.num_subcores
    nw = nc * ns
    rpw = n // nw
    c = 128
    nch = rpw // c
    ks = k // ns
    idx2 = idx.reshape(n // c, c)

    mesh = plsc.VectorSubcoreMesh(core_axis_name="c", subcore_axis_name="s")

    @functools.partial(
        pl.kernel, mesh=mesh,
        out_type=[jax.ShapeDtypeStruct((n, d), jnp.float32),
                  jax.ShapeDtypeStruct((nc, k, 16), jnp.float32)],
        scratch_types=[
            pltpu.VMEM((nch, c), jnp.int32),
            pltpu.VMEM((c, d), jnp.float32),
            pltpu.VMEM((c, 16), jnp.float32),
            pltpu.VMEM_SHARED((k, 16), jnp.float32),
            pltpu.SemaphoreType.DMA,
        ],
    )
    def sck(cb_hbm, idx_hbm, z_hbm, q_hbm, cnt_hbm, idx_v, rows_a,
            ones_v, cnt_sh, gs_a):
        cid = lax.axis_index("c")
        sid = lax.axis_index("s")
        wid = sid * nc + cid
        one = jnp.full((16,), 1.0, jnp.float32)

        def fill_ones(i, _):
            ones_v[i, :] = one
            return 0

        lax.fori_loop(0, c, fill_ones, 0)
        pltpu.sync_copy(idx_hbm.at[pl.ds(wid * nch, nch)], idx_v)
        pltpu.sync_copy(z_hbm.at[pl.ds(sid * ks, ks)],
                        cnt_sh.at[pl.ds(sid * ks, ks)])
        plsc.subcore_barrier()
        for j in range(nch):
            pltpu.async_copy(cb_hbm.at[idx_v.at[j]], rows_a, gs_a).wait()
            pltpu.sync_copy(rows_a, q_hbm.at[pl.ds((wid * nch + j) * c, c)])
            pltpu.sync_copy(ones_v, cnt_sh.at[idx_v.at[j]], add=True)
        plsc.subcore_barrier()
        pltpu.sync_copy(cnt_sh.at[pl.ds(sid * ks, ks)],
                        cnt_hbm.at[cid, pl.ds(sid * ks, ks)])

    q, cnt = sck(codebook, idx2, zeros_hist)
    counts = cnt[0, :, 0] + cnt[1, :, 0]
    return q, counts


def kernel(inputs, codebook):
    input_shape = inputs.shape
    d = input_shape[-1]
    k = codebook.shape[0]
    flat = inputs.reshape(-1, d)
    n = flat.shape[0]

    xn = jnp.sum(flat ** 2, axis=1, keepdims=True)
    cn = jnp.sum(codebook ** 2, axis=1)
    ct = codebook.T

    idx, mind = _distance_argmin(flat, xn, ct, cn.reshape(1, k))

    zeros_hist = jnp.zeros((k, 16), jnp.float32)
    quantized, counts = _sc_gather_hist(codebook, idx, zeros_hist)

    loss = COMMITMENT_COST * (jnp.sum(mind) / (n * d))
    quantized_st = (flat + (quantized - flat)).reshape(input_shape)
    avg_probs = counts / n
    perplexity = jnp.exp(-jnp.sum(avg_probs * jnp.log(avg_probs + 1e-10)))
    return quantized_st, loss, perplexity, idx


# speedup vs baseline: 1.2441x; 1.0251x over previous
import functools

import jax
import jax.numpy as jnp
from jax import lax
from jax.experimental import pallas as pl
from jax.experimental.pallas import tpu as pltpu
from jax.experimental.pallas import tpu_sc as plsc

COMMITMENT_COST = 0.25


def _segments(k):
    if k == 8192:
        return (0, 2736, 5472, 8192)
    return (0, k)


def _bf16(v):
    return v.astype(jnp.bfloat16).astype(jnp.float32)


def _argmin_body(segs, bn, x_ref, xn_ref, ct_ref, cn_ref, idx_ref, mind_ref):
    x = x_ref[...]
    x2 = x + x
    xn = xn_ref[...]
    bm = x.shape[0]
    k = ct_ref.shape[1]
    nchunk = k // bn
    nseg = len(segs) - 1
    best = [jnp.full((bm, 1), jnp.inf, jnp.float32) for _ in range(nseg)]
    bidx = [jnp.zeros((bm, 1), jnp.float32) for _ in range(nseg)]
    io = lax.broadcasted_iota(jnp.int32, (bm, bn), 1).astype(jnp.float32)
    io1 = lax.broadcasted_iota(jnp.int32, (1, bn), 1)
    for j in range(nchunk):
        lo, hi = j * bn, (j + 1) * bn
        ct = ct_ref[:, lo:hi]
        s2 = lax.dot_general(x2, ct, (((1,), (0,)), ((), ())),
                             preferred_element_type=jnp.float32)
        cn = cn_ref[:, lo:hi]
        d = (xn + cn) - s2
        for si in range(nseg):
            slo, shi = segs[si], segs[si + 1]
            if shi <= lo or slo >= hi:
                continue
            if slo <= lo and hi <= shi:
                dm = d
            else:
                gio = io1 + lo
                inseg = (gio >= slo) & (gio < shi)
                bias = jnp.where(inseg, 0.0, jnp.inf)
                dm = d + bias
            m = jnp.min(dm, axis=1, keepdims=True)
            cand = jnp.min(jnp.where(dm == m, io, jnp.float32(3e9)),
                           axis=1, keepdims=True) + jnp.float32(lo)
            take = m < best[si]
            bidx[si] = jnp.where(take, cand, bidx[si])
            best[si] = jnp.where(take, m, best[si])
    acc_cmp = _bf16(best[0])
    acc_idx = bidx[0]
    acc_true = best[0]
    for si in range(1, nseg):
        take = best[si] < acc_cmp
        acc_idx = jnp.where(take, bidx[si], acc_idx)
        acc_true = jnp.where(take, best[si], acc_true)
        acc_cmp = _bf16(jnp.where(take, best[si], acc_cmp))
    idx_ref[...] = acc_idx.astype(jnp.int32)
    mind_ref[...] = acc_true


def _distance_argmin(flat, xn, ct, cn, bm=2048, bn=1024):
    n, d = flat.shape
    k = ct.shape[1]
    segs = _segments(k)
    grid = (n // bm,)
    body = functools.partial(_argmin_body, segs, bn)
    idx2, mind = pl.pallas_call(
        body,
        grid=grid,
        in_specs=[
            pl.BlockSpec((bm, d), lambda i: (i, 0)),
            pl.BlockSpec((bm, 1), lambda i: (i, 0)),
            pl.BlockSpec((d, k), lambda i: (0, 0)),
            pl.BlockSpec((1, k), lambda i: (0, 0)),
        ],
        out_specs=[
            pl.BlockSpec((bm, 1), lambda i: (i, 0)),
            pl.BlockSpec((bm, 1), lambda i: (i, 0)),
        ],
        out_shape=[
            jax.ShapeDtypeStruct((n, 1), jnp.int32),
            jax.ShapeDtypeStruct((n, 1), jnp.float32),
        ],
        compiler_params=pltpu.CompilerParams(
            dimension_semantics=("parallel",),
        ),
    )(flat, xn, ct, cn)
    return idx2[:, 0], mind[:, 0]


def _sc_gather_hist(codebook, idx, zeros_hist):
    k, d = codebook.shape
    n = idx.shape[0]
    info = plsc.get_sparse_core_info()
    nc, ns = info.num_cores, info.num_subcores
    nw = nc * ns
    rpw = n // nw
    c = 128
    nch = rpw // c
    ks = k // ns
    idx2 = idx.reshape(n // c, c)

    mesh = plsc.VectorSubcoreMesh(core_axis_name="c", subcore_axis_name="s")

    @functools.partial(
        pl.kernel, mesh=mesh,
        out_type=[jax.ShapeDtypeStruct((n, d), jnp.float32),
                  jax.ShapeDtypeStruct((nc, k, 16), jnp.float32)],
        scratch_types=[
            pltpu.VMEM((nch, c), jnp.int32),
            pltpu.VMEM((c, d), jnp.float32),
            pltpu.VMEM((c, 16), jnp.float32),
            pltpu.VMEM_SHARED((k, 16), jnp.float32),
            pltpu.SemaphoreType.DMA,
        ],
    )
    def sck(cb_hbm, idx_hbm, z_hbm, q_hbm, cnt_hbm, idx_v, rows_a,
            ones_v, cnt_sh, gs_a):
        cid = lax.axis_index("c")
        sid = lax.axis_index("s")
        wid = sid * nc + cid
        one = jnp.full((16,), 1.0, jnp.float32)

        def fill_ones(i, _):
            ones_v[i, :] = one
            return 0

        lax.fori_loop(0, c, fill_ones, 0)
        pltpu.sync_copy(idx_hbm.at[pl.ds(wid * nch, nch)], idx_v)
        pltpu.sync_copy(z_hbm.at[pl.ds(sid * ks, ks)],
                        cnt_sh.at[pl.ds(sid * ks, ks)])
        plsc.subcore_barrier()
        for j in range(nch):
            pltpu.async_copy(cb_hbm.at[idx_v.at[j]], rows_a, gs_a).wait()
            pltpu.sync_copy(rows_a, q_hbm.at[pl.ds((wid * nch + j) * c, c)])
            pltpu.sync_copy(ones_v, cnt_sh.at[idx_v.at[j]], add=True)
        plsc.subcore_barrier()
        pltpu.sync_copy(cnt_sh.at[pl.ds(sid * ks, ks)],
                        cnt_hbm.at[cid, pl.ds(sid * ks, ks)])

    q, cnt = sck(codebook, idx2, zeros_hist)
    counts = cnt[0, :, 0] + cnt[1, :, 0]
    return q, counts


def kernel(inputs, codebook):
    input_shape = inputs.shape
    d = input_shape[-1]
    k = codebook.shape[0]
    flat = inputs.reshape(-1, d)
    n = flat.shape[0]

    xn = jnp.sum(flat ** 2, axis=1, keepdims=True)
    cn = jnp.sum(codebook ** 2, axis=1)
    ct = codebook.T

    idx, mind = _distance_argmin(flat, xn, ct, cn.reshape(1, k))

    zeros_hist = jnp.zeros((k, 16), jnp.float32)
    quantized, counts = _sc_gather_hist(codebook, idx, zeros_hist)

    loss = COMMITMENT_COST * (jnp.sum(mind) / (n * d))
    quantized_st = (flat + (quantized - flat)).reshape(input_shape)
    avg_probs = counts / n
    perplexity = jnp.exp(-jnp.sum(avg_probs * jnp.log(avg_probs + 1e-10)))
    return quantized_st, loss, perplexity, idx
